# baseline (speedup 1.0000x reference)
import functools

import jax
import jax.numpy as jnp
from jax import lax
from jax.experimental import pallas as pl
from jax.experimental.pallas import tpu as pltpu
from jax.experimental.pallas import tpu_sc as plsc

EPS = 1e-6
T = 128
TA = 256
NC = 2
NS = 16
NW = NC * NS


def _router_body(x_ref, nw_ref, gw_ref, xn_ref, logits_ref, sel_ref,
                 wr_ref, counts_ref, prior_ref):
    k = pl.program_id(0)

    @pl.when(k == 0)
    def _():
        prior_ref[...] = jnp.zeros_like(prior_ref)

    x = x_ref[...]
    v = jnp.mean(x * x, axis=-1, keepdims=True)
    xn = (x * lax.rsqrt(v + EPS)) * nw_ref[...]
    xn_ref[...] = xn
    logits = lax.dot_general(
        xn, gw_ref[...], (((1,), (1,)), ((), ())),
        preferred_element_type=jnp.float32)
    logits_ref[...] = logits
    e = logits.shape[-1]
    m = jnp.max(logits, axis=-1, keepdims=True)
    iota = lax.broadcasted_iota(jnp.int32, logits.shape, 1)
    am = jnp.min(jnp.where(logits == m, iota, e), axis=-1, keepdims=True)
    sel_ref[...] = jnp.broadcast_to(am, sel_ref.shape)
    ta = am.shape[0]
    oh = (lax.broadcasted_iota(jnp.int32, (ta, 128), 1) == am
          ).astype(jnp.float32)
    r2 = lax.broadcasted_iota(jnp.int32, (ta, ta), 0)
    c2 = lax.broadcasted_iota(jnp.int32, (ta, ta), 1)
    tri = (r2 > c2).astype(jnp.float32)
    excl = jnp.dot(tri, oh, preferred_element_type=jnp.float32)
    prior = prior_ref[...]
    wr = jnp.sum((excl + prior) * oh, axis=1, keepdims=True)
    wr_ref[...] = jnp.broadcast_to(wr.astype(jnp.int32), wr_ref.shape)
    new_prior = prior + jnp.sum(oh, axis=0, keepdims=True)
    prior_ref[...] = new_prior
    counts_ref[...] = jnp.broadcast_to(new_prior, counts_ref.shape)


def _ffn_body(ue_ref, ut_ref, urs_ref, ure_ref, uf_ref,
              xs_ref, w1_ref, w3_ref, w2_ref, out_ref):
    u = pl.program_id(0)

    @pl.when(uf_ref[u] == 1)
    def _():
        out_ref[...] = jnp.zeros_like(out_ref)

    x = xs_ref[...]
    h1 = jnp.dot(x, w1_ref[0], preferred_element_type=jnp.float32)
    h3 = jnp.dot(x, w3_ref[0], preferred_element_type=jnp.float32)
    h = (h1 * jax.nn.sigmoid(h1)) * h3
    y = jnp.dot(h, w2_ref[0], preferred_element_type=jnp.float32)
    rows = lax.broadcasted_iota(jnp.int32, y.shape, 0)
    mask = (rows >= urs_ref[u]) & (rows < ure_ref[u])
    out_ref[...] += jnp.where(mask, y, 0.0)


def _unit_metadata(counts, n_experts, n_tokens):
    nt = n_tokens // T
    n_units = nt + n_experts - 1
    ends = jnp.cumsum(counts)
    offs = ends - counts
    first_t = offs // T
    last_t = jnp.where(counts > 0, (ends - 1) // T, first_t - 1)
    span = jnp.where(counts > 0, last_t - first_t + 1, 0)
    cum = jnp.cumsum(span)
    ubase = cum - span
    total = cum[-1]
    u = jnp.arange(n_units, dtype=jnp.int32)
    ue = jnp.minimum(
        jnp.searchsorted(cum, u, side="right").astype(jnp.int32),
        n_experts - 1)
    valid = u < total
    nonempty = counts > 0
    pad_e = (n_experts - 1
             - jnp.argmax(nonempty[::-1]).astype(jnp.int32))
    ue = jnp.where(valid, ue, pad_e)
    tab = jnp.stack([first_t, ubase, offs, ends])
    g = jnp.take(tab, ue, axis=1)
    ut = jnp.minimum(g[0] + (u - g[1]), nt - 1).astype(jnp.int32)
    urs = jnp.maximum(g[2] - ut * T, 0)
    ure = jnp.where(valid, jnp.minimum(g[3] - ut * T, T), 0)
    uf = jnp.concatenate(
        [jnp.ones((1,), jnp.int32), (ut[1:] != ut[:-1]).astype(jnp.int32)])
    return offs, ue, urs.astype(jnp.int32), ure.astype(jnp.int32), ut, uf


def _make_sc_permute(n_tokens, d, reverse):
    rpw = n_tokens // NW
    mesh = plsc.VectorSubcoreMesh(core_axis_name="c", subcore_axis_name="s")

    @functools.partial(
        pl.kernel,
        out_type=jax.ShapeDtypeStruct((n_tokens, d), jnp.float32),
        mesh=mesh,
        scratch_types=[
            pltpu.VMEM((rpw,), jnp.int32),
            pltpu.VMEM((rpw, d), jnp.float32),
            pltpu.SemaphoreType.DMA,
        ],
    )
    def body(src_hbm, order_hbm, out_hbm, idx_v, rows_v, sem):
        wid = lax.axis_index("s") * NC + lax.axis_index("c")
        base = wid * rpw
        pltpu.sync_copy(order_hbm.at[pl.ds(base, rpw)], idx_v)
        if reverse:
            pltpu.sync_copy(src_hbm.at[pl.ds(base, rpw)], rows_v)
            pltpu.async_copy(rows_v, out_hbm.at[idx_v], sem).wait()
        else:
            pltpu.async_copy(src_hbm.at[idx_v], rows_v, sem).wait()
            pltpu.sync_copy(rows_v, out_hbm.at[pl.ds(base, rpw)])

    return body


def kernel(hidden_states, norm_w, gate_w, w1, w3, w2):
    b, s, d = hidden_states.shape
    n_experts, _, dff = w1.shape
    n_tokens = b * s
    nt = n_tokens // T
    n_units = nt + n_experts - 1
    x2d = hidden_states.reshape(n_tokens, d)

    xn, logits, selb, wrb, countsb = pl.pallas_call(
        _router_body,
        grid=(n_tokens // TA,),
        in_specs=[
            pl.BlockSpec((TA, d), lambda i: (i, 0)),
            pl.BlockSpec((1, d), lambda i: (0, 0)),
            pl.BlockSpec((n_experts, d), lambda i: (0, 0)),
        ],
        out_specs=[
            pl.BlockSpec((TA, d), lambda i: (i, 0)),
            pl.BlockSpec((TA, n_experts), lambda i: (i, 0)),
            pl.BlockSpec((TA, 128), lambda i: (i, 0)),
            pl.BlockSpec((TA, 128), lambda i: (i, 0)),
            pl.BlockSpec((8, 128), lambda i: (0, 0)),
        ],
        out_shape=[
            jax.ShapeDtypeStruct((n_tokens, d), jnp.float32),
            jax.ShapeDtypeStruct((n_tokens, n_experts), jnp.float32),
            jax.ShapeDtypeStruct((n_tokens, 128), jnp.int32),
            jax.ShapeDtypeStruct((n_tokens, 128), jnp.int32),
            jax.ShapeDtypeStruct((8, 128), jnp.float32),
        ],
        scratch_shapes=[pltpu.VMEM((1, 128), jnp.float32)],
    )(x2d, norm_w.reshape(1, d), gate_w)
    sel = selb[:, 0]
    counts = countsb[0, :n_experts].astype(jnp.int32)

    offs, ue, urs, ure, ut, uf = _unit_metadata(counts, n_experts, n_tokens)
    rank = (jnp.take(offs, sel) + wrb[:, 0]).astype(jnp.int32)

    xs = _make_sc_permute(n_tokens, d, reverse=True)(xn, rank)

    grid_spec = pltpu.PrefetchScalarGridSpec(
        num_scalar_prefetch=5,
        grid=(n_units,),
        in_specs=[
            pl.BlockSpec((T, d), lambda u, ue, ut, urs, ure, uf: (ut[u], 0)),
            pl.BlockSpec(
                (1, d, dff), lambda u, ue, ut, urs, ure, uf: (ue[u], 0, 0)),
            pl.BlockSpec(
                (1, d, dff), lambda u, ue, ut, urs, ure, uf: (ue[u], 0, 0)),
            pl.BlockSpec(
                (1, dff, d), lambda u, ue, ut, urs, ure, uf: (ue[u], 0, 0)),
        ],
        out_specs=pl.BlockSpec(
            (T, d), lambda u, ue, ut, urs, ure, uf: (ut[u], 0)),
    )
    ys = pl.pallas_call(
        _ffn_body,
        grid_spec=grid_spec,
        out_shape=jax.ShapeDtypeStruct((n_tokens, d), jnp.float32),
    )(ue, ut, urs, ure, uf, xs, w1, w3, w2)

    final = _make_sc_permute(n_tokens, d, reverse=False)(ys, rank)
    return final.reshape(b, s, d), logits


# speedup vs baseline: 1.1723x; 1.1723x over previous
import functools

import jax
import jax.numpy as jnp
from jax import lax
from jax.experimental import pallas as pl
from jax.experimental.pallas import tpu as pltpu
from jax.experimental.pallas import tpu_sc as plsc

EPS = 1e-6
T = 128
TA = 256
NC = 2
NS = 16
NW = NC * NS


def _router_body(x_ref, nw_ref, gw_ref, xn_ref, logits_ref, sel_ref):
    x = x_ref[...]
    v = jnp.mean(x * x, axis=-1, keepdims=True)
    xn = (x * lax.rsqrt(v + EPS)) * nw_ref[...]
    xn_ref[...] = xn
    logits = lax.dot_general(
        xn, gw_ref[...], (((1,), (1,)), ((), ())),
        preferred_element_type=jnp.float32)
    logits_ref[...] = logits
    e = logits.shape[-1]
    m = jnp.max(logits, axis=-1, keepdims=True)
    iota = lax.broadcasted_iota(jnp.int32, logits.shape, 1)
    am = jnp.min(jnp.where(logits == m, iota, e), axis=-1, keepdims=True)
    sel_ref[...] = jnp.broadcast_to(am, sel_ref.shape)


def _ffn_body(ue_ref, ut_ref, urs_ref, ure_ref, uf_ref,
              xs_ref, w1_ref, w3_ref, w2_ref, out_ref):
    u = pl.program_id(0)

    @pl.when(uf_ref[u] == 1)
    def _():
        out_ref[...] = jnp.zeros_like(out_ref)

    x = xs_ref[...]
    h1 = jnp.dot(x, w1_ref[0], preferred_element_type=jnp.float32)
    h3 = jnp.dot(x, w3_ref[0], preferred_element_type=jnp.float32)
    h = (h1 * jax.nn.sigmoid(h1)) * h3
    y = jnp.dot(h, w2_ref[0], preferred_element_type=jnp.float32)
    rows = lax.broadcasted_iota(jnp.int32, y.shape, 0)
    mask = (rows >= urs_ref[u]) & (rows < ure_ref[u])
    out_ref[...] += jnp.where(mask, y, 0.0)


def _unit_metadata(sel, n_experts, n_tokens):
    nt = n_tokens // T
    n_units = nt + n_experts - 1
    order = jnp.argsort(sel, stable=True).astype(jnp.int32)
    counts = jnp.bincount(sel, length=n_experts).astype(jnp.int32)
    ends = jnp.cumsum(counts)
    offs = ends - counts
    first_t = offs // T
    last_t = jnp.where(counts > 0, (ends - 1) // T, first_t - 1)
    span = jnp.where(counts > 0, last_t - first_t + 1, 0)
    cum = jnp.cumsum(span)
    ubase = cum - span
    total = cum[-1]
    u = jnp.arange(n_units, dtype=jnp.int32)
    ue = jnp.minimum(
        jnp.sum((cum[None, :] <= u[:, None]).astype(jnp.int32), axis=1),
        n_experts - 1)
    valid = u < total
    e_ids = jnp.arange(n_experts, dtype=jnp.int32)
    pad_e = jnp.max(jnp.where(counts > 0, e_ids, 0))
    ue = jnp.where(valid, ue, pad_e)
    tab = jnp.stack([first_t, ubase, offs, ends])
    g = jnp.take(tab, ue, axis=1)
    ut = jnp.minimum(g[0] + (u - g[1]), nt - 1).astype(jnp.int32)
    urs = jnp.maximum(g[2] - ut * T, 0)
    ure = jnp.where(valid, jnp.minimum(g[3] - ut * T, T), 0)
    uf = jnp.concatenate(
        [jnp.ones((1,), jnp.int32), (ut[1:] != ut[:-1]).astype(jnp.int32)])
    return order, ue, urs.astype(jnp.int32), ure.astype(jnp.int32), ut, uf


def _make_sc_permute(n_tokens, d, reverse):
    rpw = n_tokens // NW
    mesh = plsc.VectorSubcoreMesh(core_axis_name="c", subcore_axis_name="s")

    @functools.partial(
        pl.kernel,
        out_type=jax.ShapeDtypeStruct((n_tokens, d), jnp.float32),
        mesh=mesh,
        scratch_types=[
            pltpu.VMEM((rpw,), jnp.int32),
            pltpu.VMEM((rpw, d), jnp.float32),
            pltpu.SemaphoreType.DMA,
        ],
    )
    def body(src_hbm, order_hbm, out_hbm, idx_v, rows_v, sem):
        wid = lax.axis_index("s") * NC + lax.axis_index("c")
        base = wid * rpw
        pltpu.sync_copy(order_hbm.at[pl.ds(base, rpw)], idx_v)
        if reverse:
            pltpu.sync_copy(src_hbm.at[pl.ds(base, rpw)], rows_v)
            pltpu.async_copy(rows_v, out_hbm.at[idx_v], sem).wait()
        else:
            pltpu.async_copy(src_hbm.at[idx_v], rows_v, sem).wait()
            pltpu.sync_copy(rows_v, out_hbm.at[pl.ds(base, rpw)])

    return body


def kernel(hidden_states, norm_w, gate_w, w1, w3, w2):
    b, s, d = hidden_states.shape
    n_experts, _, dff = w1.shape
    n_tokens = b * s
    nt = n_tokens // T
    n_units = nt + n_experts - 1
    x2d = hidden_states.reshape(n_tokens, d)

    xn, logits, selb = pl.pallas_call(
        _router_body,
        grid=(n_tokens // TA,),
        in_specs=[
            pl.BlockSpec((TA, d), lambda i: (i, 0)),
            pl.BlockSpec((1, d), lambda i: (0, 0)),
            pl.BlockSpec((n_experts, d), lambda i: (0, 0)),
        ],
        out_specs=[
            pl.BlockSpec((TA, d), lambda i: (i, 0)),
            pl.BlockSpec((TA, n_experts), lambda i: (i, 0)),
            pl.BlockSpec((TA, 128), lambda i: (i, 0)),
        ],
        out_shape=[
            jax.ShapeDtypeStruct((n_tokens, d), jnp.float32),
            jax.ShapeDtypeStruct((n_tokens, n_experts), jnp.float32),
            jax.ShapeDtypeStruct((n_tokens, 128), jnp.int32),
        ],
    )(x2d, norm_w.reshape(1, d), gate_w)
    sel = selb[:, 0]

    order, ue, urs, ure, ut, uf = _unit_metadata(sel, n_experts, n_tokens)

    xs = _make_sc_permute(n_tokens, d, reverse=False)(xn, order)

    grid_spec = pltpu.PrefetchScalarGridSpec(
        num_scalar_prefetch=5,
        grid=(n_units,),
        in_specs=[
            pl.BlockSpec((T, d), lambda u, ue, ut, urs, ure, uf: (ut[u], 0)),
            pl.BlockSpec(
                (1, d, dff), lambda u, ue, ut, urs, ure, uf: (ue[u], 0, 0)),
            pl.BlockSpec(
                (1, d, dff), lambda u, ue, ut, urs, ure, uf: (ue[u], 0, 0)),
            pl.BlockSpec(
                (1, dff, d), lambda u, ue, ut, urs, ure, uf: (ue[u], 0, 0)),
        ],
        out_specs=pl.BlockSpec(
            (T, d), lambda u, ue, ut, urs, ure, uf: (ut[u], 0)),
    )
    ys = pl.pallas_call(
        _ffn_body,
        grid_spec=grid_spec,
        out_shape=jax.ShapeDtypeStruct((n_tokens, d), jnp.float32),
    )(ue, ut, urs, ure, uf, xs, w1, w3, w2)

    final = _make_sc_permute(n_tokens, d, reverse=True)(ys, order)
    return final.reshape(b, s, d), logits


# speedup vs baseline: 1.1956x; 1.0199x over previous
import functools

import jax
import jax.numpy as jnp
from jax import lax
from jax.experimental import pallas as pl
from jax.experimental.pallas import tpu as pltpu
from jax.experimental.pallas import tpu_sc as plsc

EPS = 1e-6
T = 128
TA = 256
NC = 2
NS = 16
NW = NC * NS


def _router_body(x_ref, nw_ref, gw_ref, xn_ref, logits_ref, sel_ref):
    x = x_ref[...]
    v = jnp.mean(x * x, axis=-1, keepdims=True)
    xn = (x * lax.rsqrt(v + EPS)) * nw_ref[...]
    xn_ref[...] = xn
    logits = lax.dot_general(
        xn, gw_ref[...], (((1,), (1,)), ((), ())),
        preferred_element_type=jnp.float32)
    logits_ref[...] = logits
    e = logits.shape[-1]
    m = jnp.max(logits, axis=-1, keepdims=True)
    iota = lax.broadcasted_iota(jnp.int32, logits.shape, 1)
    am = jnp.min(jnp.where(logits == m, iota, e), axis=-1, keepdims=True)
    sel_ref[...] = jnp.broadcast_to(am, sel_ref.shape)


def _ffn_body(ue_ref, ut_ref, urs_ref, ure_ref, uf_ref,
              xs_ref, w1_ref, w3_ref, w2_ref, out_ref):
    u = pl.program_id(0)

    @pl.when(uf_ref[u] == 1)
    def _():
        out_ref[...] = jnp.zeros_like(out_ref)

    x = xs_ref[...]
    h1 = jnp.dot(x, w1_ref[0], preferred_element_type=jnp.float32)
    h3 = jnp.dot(x, w3_ref[0], preferred_element_type=jnp.float32)
    h = (h1 * jax.nn.sigmoid(h1)) * h3
    y = jnp.dot(h, w2_ref[0], preferred_element_type=jnp.float32)
    rows = lax.broadcasted_iota(jnp.int32, y.shape, 0)
    mask = (rows >= urs_ref[u]) & (rows < ure_ref[u])
    out_ref[...] += jnp.where(mask, y, 0.0)


def _unit_metadata(sel, n_experts, n_tokens):
    nt = n_tokens // T
    n_units = nt + n_experts - 1
    ssel, order = lax.sort_key_val(
        sel, jnp.arange(n_tokens, dtype=jnp.int32), is_stable=True)
    e_ids0 = jnp.arange(n_experts, dtype=jnp.int32)
    counts = jnp.sum(
        (ssel[:, None] == e_ids0[None, :]).astype(jnp.int32), axis=0)
    ends = jnp.cumsum(counts)
    offs = ends - counts
    first_t = offs // T
    last_t = jnp.where(counts > 0, (ends - 1) // T, first_t - 1)
    span = jnp.where(counts > 0, last_t - first_t + 1, 0)
    cum = jnp.cumsum(span)
    ubase = cum - span
    total = cum[-1]
    u = jnp.arange(n_units, dtype=jnp.int32)
    ue = jnp.minimum(
        jnp.sum((cum[None, :] <= u[:, None]).astype(jnp.int32), axis=1),
        n_experts - 1)
    valid = u < total
    e_ids = jnp.arange(n_experts, dtype=jnp.int32)
    pad_e = jnp.max(jnp.where(counts > 0, e_ids, 0))
    ue = jnp.where(valid, ue, pad_e)
    tab = jnp.stack([first_t, ubase, offs, ends])
    g = jnp.take(tab, ue, axis=1)
    ut = jnp.minimum(g[0] + (u - g[1]), nt - 1).astype(jnp.int32)
    urs = jnp.maximum(g[2] - ut * T, 0)
    ure = jnp.where(valid, jnp.minimum(g[3] - ut * T, T), 0)
    uf = jnp.concatenate(
        [jnp.ones((1,), jnp.int32), (ut[1:] != ut[:-1]).astype(jnp.int32)])
    return order, ue, urs.astype(jnp.int32), ure.astype(jnp.int32), ut, uf


def _make_sc_permute(n_tokens, d, reverse):
    rpw = n_tokens // NW
    mesh = plsc.VectorSubcoreMesh(core_axis_name="c", subcore_axis_name="s")

    @functools.partial(
        pl.kernel,
        out_type=jax.ShapeDtypeStruct((n_tokens, d), jnp.float32),
        mesh=mesh,
        scratch_types=[
            pltpu.VMEM((rpw,), jnp.int32),
            pltpu.VMEM((rpw, d), jnp.float32),
            pltpu.SemaphoreType.DMA,
        ],
    )
    def body(src_hbm, order_hbm, out_hbm, idx_v, rows_v, sem):
        wid = lax.axis_index("s") * NC + lax.axis_index("c")
        base = wid * rpw
        pltpu.sync_copy(order_hbm.at[pl.ds(base, rpw)], idx_v)
        if reverse:
            pltpu.sync_copy(src_hbm.at[pl.ds(base, rpw)], rows_v)
            pltpu.async_copy(rows_v, out_hbm.at[idx_v], sem).wait()
        else:
            pltpu.async_copy(src_hbm.at[idx_v], rows_v, sem).wait()
            pltpu.sync_copy(rows_v, out_hbm.at[pl.ds(base, rpw)])

    return body


def kernel(hidden_states, norm_w, gate_w, w1, w3, w2):
    b, s, d = hidden_states.shape
    n_experts, _, dff = w1.shape
    n_tokens = b * s
    nt = n_tokens // T
    n_units = nt + n_experts - 1
    x2d = hidden_states.reshape(n_tokens, d)

    xn, logits, selb = pl.pallas_call(
        _router_body,
        grid=(n_tokens // TA,),
        in_specs=[
            pl.BlockSpec((TA, d), lambda i: (i, 0)),
            pl.BlockSpec((1, d), lambda i: (0, 0)),
            pl.BlockSpec((n_experts, d), lambda i: (0, 0)),
        ],
        out_specs=[
            pl.BlockSpec((TA, d), lambda i: (i, 0)),
            pl.BlockSpec((TA, n_experts), lambda i: (i, 0)),
            pl.BlockSpec((TA, 128), lambda i: (i, 0)),
        ],
        out_shape=[
            jax.ShapeDtypeStruct((n_tokens, d), jnp.float32),
            jax.ShapeDtypeStruct((n_tokens, n_experts), jnp.float32),
            jax.ShapeDtypeStruct((n_tokens, 128), jnp.int32),
        ],
    )(x2d, norm_w.reshape(1, d), gate_w)
    sel = selb[:, 0]

    order, ue, urs, ure, ut, uf = _unit_metadata(sel, n_experts, n_tokens)

    xs = _make_sc_permute(n_tokens, d, reverse=False)(xn, order)

    grid_spec = pltpu.PrefetchScalarGridSpec(
        num_scalar_prefetch=5,
        grid=(n_units,),
        in_specs=[
            pl.BlockSpec((T, d), lambda u, ue, ut, urs, ure, uf: (ut[u], 0)),
            pl.BlockSpec(
                (1, d, dff), lambda u, ue, ut, urs, ure, uf: (ue[u], 0, 0)),
            pl.BlockSpec(
                (1, d, dff), lambda u, ue, ut, urs, ure, uf: (ue[u], 0, 0)),
            pl.BlockSpec(
                (1, dff, d), lambda u, ue, ut, urs, ure, uf: (ue[u], 0, 0)),
        ],
        out_specs=pl.BlockSpec(
            (T, d), lambda u, ue, ut, urs, ure, uf: (ut[u], 0)),
    )
    ys = pl.pallas_call(
        _ffn_body,
        grid_spec=grid_spec,
        out_shape=jax.ShapeDtypeStruct((n_tokens, d), jnp.float32),
    )(ue, ut, urs, ure, uf, xs, w1, w3, w2)

    final = _make_sc_permute(n_tokens, d, reverse=True)(ys, order)
    return final.reshape(b, s, d), logits


# speedup vs baseline: 1.2448x; 1.0412x over previous
import functools

import jax
import jax.numpy as jnp
from jax import lax
from jax.experimental import pallas as pl
from jax.experimental.pallas import tpu as pltpu
from jax.experimental.pallas import tpu_sc as plsc

EPS = 1e-6
T = 256
TA = 256
NC = 2
NS = 16
NW = NC * NS


def _router_body(x_ref, nw_ref, gw_ref, xn_ref, logits_ref, sel_ref):
    x = x_ref[...]
    v = jnp.mean(x * x, axis=-1, keepdims=True)
    xn = (x * lax.rsqrt(v + EPS)) * nw_ref[...]
    xn_ref[...] = xn
    logits = lax.dot_general(
        xn, gw_ref[...], (((1,), (1,)), ((), ())),
        preferred_element_type=jnp.float32)
    logits_ref[...] = logits
    e = logits.shape[-1]
    m = jnp.max(logits, axis=-1, keepdims=True)
    iota = lax.broadcasted_iota(jnp.int32, logits.shape, 1)
    am = jnp.min(jnp.where(logits == m, iota, e), axis=-1, keepdims=True)
    sel_ref[...] = jnp.broadcast_to(am, sel_ref.shape)


def _ffn_body(ue_ref, ut_ref, urs_ref, ure_ref, uf_ref,
              xs_ref, w1_ref, w3_ref, w2_ref, out_ref):
    u = pl.program_id(0)

    @pl.when(uf_ref[u] == 1)
    def _():
        out_ref[...] = jnp.zeros_like(out_ref)

    x = xs_ref[...]
    h1 = jnp.dot(x, w1_ref[0], preferred_element_type=jnp.float32)
    h3 = jnp.dot(x, w3_ref[0], preferred_element_type=jnp.float32)
    h = (h1 * jax.nn.sigmoid(h1)) * h3
    y = jnp.dot(h, w2_ref[0], preferred_element_type=jnp.float32)
    rows = lax.broadcasted_iota(jnp.int32, y.shape, 0)
    mask = (rows >= urs_ref[u]) & (rows < ure_ref[u])
    out_ref[...] += jnp.where(mask, y, 0.0)


def _unit_metadata(sel, n_experts, n_tokens):
    nt = n_tokens // T
    n_units = nt + n_experts - 1
    ssel, order = lax.sort_key_val(
        sel, jnp.arange(n_tokens, dtype=jnp.int32), is_stable=True)
    e_ids0 = jnp.arange(n_experts, dtype=jnp.int32)
    counts = jnp.sum(
        (ssel[:, None] == e_ids0[None, :]).astype(jnp.int32), axis=0)
    ends = jnp.cumsum(counts)
    offs = ends - counts
    first_t = offs // T
    last_t = jnp.where(counts > 0, (ends - 1) // T, first_t - 1)
    span = jnp.where(counts > 0, last_t - first_t + 1, 0)
    cum = jnp.cumsum(span)
    ubase = cum - span
    total = cum[-1]
    u = jnp.arange(n_units, dtype=jnp.int32)
    ue = jnp.minimum(
        jnp.sum((cum[None, :] <= u[:, None]).astype(jnp.int32), axis=1),
        n_experts - 1)
    valid = u < total
    e_ids = jnp.arange(n_experts, dtype=jnp.int32)
    pad_e = jnp.max(jnp.where(counts > 0, e_ids, 0))
    ue = jnp.where(valid, ue, pad_e)
    tab = jnp.stack([first_t, ubase, offs, ends])
    g = jnp.take(tab, ue, axis=1)
    ut = jnp.minimum(g[0] + (u - g[1]), nt - 1).astype(jnp.int32)
    urs = jnp.maximum(g[2] - ut * T, 0)
    ure = jnp.where(valid, jnp.minimum(g[3] - ut * T, T), 0)
    uf = jnp.concatenate(
        [jnp.ones((1,), jnp.int32), (ut[1:] != ut[:-1]).astype(jnp.int32)])
    return order, ue, urs.astype(jnp.int32), ure.astype(jnp.int32), ut, uf


def _make_sc_permute(n_tokens, d, reverse):
    rpw = n_tokens // NW
    mesh = plsc.VectorSubcoreMesh(core_axis_name="c", subcore_axis_name="s")

    @functools.partial(
        pl.kernel,
        out_type=jax.ShapeDtypeStruct((n_tokens, d), jnp.float32),
        mesh=mesh,
        scratch_types=[
            pltpu.VMEM((rpw,), jnp.int32),
            pltpu.VMEM((rpw, d), jnp.float32),
            pltpu.SemaphoreType.DMA,
        ],
    )
    def body(src_hbm, order_hbm, out_hbm, idx_v, rows_v, sem):
        wid = lax.axis_index("s") * NC + lax.axis_index("c")
        base = wid * rpw
        pltpu.sync_copy(order_hbm.at[pl.ds(base, rpw)], idx_v)
        if reverse:
            pltpu.sync_copy(src_hbm.at[pl.ds(base, rpw)], rows_v)
            pltpu.async_copy(rows_v, out_hbm.at[idx_v], sem).wait()
        else:
            pltpu.async_copy(src_hbm.at[idx_v], rows_v, sem).wait()
            pltpu.sync_copy(rows_v, out_hbm.at[pl.ds(base, rpw)])

    return body


def kernel(hidden_states, norm_w, gate_w, w1, w3, w2):
    b, s, d = hidden_states.shape
    n_experts, _, dff = w1.shape
    n_tokens = b * s
    nt = n_tokens // T
    n_units = nt + n_experts - 1
    x2d = hidden_states.reshape(n_tokens, d)

    xn, logits, selb = pl.pallas_call(
        _router_body,
        grid=(n_tokens // TA,),
        in_specs=[
            pl.BlockSpec((TA, d), lambda i: (i, 0)),
            pl.BlockSpec((1, d), lambda i: (0, 0)),
            pl.BlockSpec((n_experts, d), lambda i: (0, 0)),
        ],
        out_specs=[
            pl.BlockSpec((TA, d), lambda i: (i, 0)),
            pl.BlockSpec((TA, n_experts), lambda i: (i, 0)),
            pl.BlockSpec((TA, 128), lambda i: (i, 0)),
        ],
        out_shape=[
            jax.ShapeDtypeStruct((n_tokens, d), jnp.float32),
            jax.ShapeDtypeStruct((n_tokens, n_experts), jnp.float32),
            jax.ShapeDtypeStruct((n_tokens, 128), jnp.int32),
        ],
    )(x2d, norm_w.reshape(1, d), gate_w)
    sel = selb[:, 0]

    order, ue, urs, ure, ut, uf = _unit_metadata(sel, n_experts, n_tokens)

    xs = _make_sc_permute(n_tokens, d, reverse=False)(xn, order)

    grid_spec = pltpu.PrefetchScalarGridSpec(
        num_scalar_prefetch=5,
        grid=(n_units,),
        in_specs=[
            pl.BlockSpec((T, d), lambda u, ue, ut, urs, ure, uf: (ut[u], 0)),
            pl.BlockSpec(
                (1, d, dff), lambda u, ue, ut, urs, ure, uf: (ue[u], 0, 0)),
            pl.BlockSpec(
                (1, d, dff), lambda u, ue, ut, urs, ure, uf: (ue[u], 0, 0)),
            pl.BlockSpec(
                (1, dff, d), lambda u, ue, ut, urs, ure, uf: (ue[u], 0, 0)),
        ],
        out_specs=pl.BlockSpec(
            (T, d), lambda u, ue, ut, urs, ure, uf: (ut[u], 0)),
    )
    ys = pl.pallas_call(
        _ffn_body,
        grid_spec=grid_spec,
        out_shape=jax.ShapeDtypeStruct((n_tokens, d), jnp.float32),
    )(ue, ut, urs, ure, uf, xs, w1, w3, w2)

    final = _make_sc_permute(n_tokens, d, reverse=True)(ys, order)
    return final.reshape(b, s, d), logits


# speedup vs baseline: 1.2537x; 1.0071x over previous
import functools

import jax
import jax.numpy as jnp
from jax import lax
from jax.experimental import pallas as pl
from jax.experimental.pallas import tpu as pltpu
from jax.experimental.pallas import tpu_sc as plsc

EPS = 1e-6
T = 256
TA = 512
NC = 2
NS = 16
NW = NC * NS


def _router_body(x_ref, nw_ref, gw_ref, xn_ref, logits_ref, sel_ref):
    x = x_ref[...]
    v = jnp.mean(x * x, axis=-1, keepdims=True)
    xn = (x * lax.rsqrt(v + EPS)) * nw_ref[...]
    xn_ref[...] = xn
    logits = lax.dot_general(
        xn, gw_ref[...], (((1,), (1,)), ((), ())),
        preferred_element_type=jnp.float32)
    logits_ref[...] = logits
    e = logits.shape[-1]
    m = jnp.max(logits, axis=-1, keepdims=True)
    iota = lax.broadcasted_iota(jnp.int32, logits.shape, 1)
    am = jnp.min(jnp.where(logits == m, iota, e), axis=-1, keepdims=True)
    sel_ref[...] = jnp.broadcast_to(am, sel_ref.shape)


def _ffn_body(ue_ref, ut_ref, urs_ref, ure_ref, uf_ref,
              xs_ref, w1_ref, w3_ref, w2_ref, out_ref):
    u = pl.program_id(0)

    @pl.when(uf_ref[u] == 1)
    def _():
        out_ref[...] = jnp.zeros_like(out_ref)

    x = xs_ref[...]
    h1 = jnp.dot(x, w1_ref[0], preferred_element_type=jnp.float32)
    h3 = jnp.dot(x, w3_ref[0], preferred_element_type=jnp.float32)
    h = (h1 * jax.nn.sigmoid(h1)) * h3
    y = jnp.dot(h, w2_ref[0], preferred_element_type=jnp.float32)
    rows = lax.broadcasted_iota(jnp.int32, y.shape, 0)
    mask = (rows >= urs_ref[u]) & (rows < ure_ref[u])
    out_ref[...] += jnp.where(mask, y, 0.0)


def _unit_metadata(sel, n_experts, n_tokens):
    nt = n_tokens // T
    n_units = nt + n_experts - 1
    ssel, order = lax.sort_key_val(
        sel, jnp.arange(n_tokens, dtype=jnp.int32), is_stable=True)
    e_ids0 = jnp.arange(n_experts, dtype=jnp.int32)
    counts = jnp.sum(
        (ssel[:, None] == e_ids0[None, :]).astype(jnp.int32), axis=0)
    ends = jnp.cumsum(counts)
    offs = ends - counts
    first_t = offs // T
    last_t = jnp.where(counts > 0, (ends - 1) // T, first_t - 1)
    span = jnp.where(counts > 0, last_t - first_t + 1, 0)
    cum = jnp.cumsum(span)
    ubase = cum - span
    total = cum[-1]
    u = jnp.arange(n_units, dtype=jnp.int32)
    ue = jnp.minimum(
        jnp.sum((cum[None, :] <= u[:, None]).astype(jnp.int32), axis=1),
        n_experts - 1)
    valid = u < total
    e_ids = jnp.arange(n_experts, dtype=jnp.int32)
    pad_e = jnp.max(jnp.where(counts > 0, e_ids, 0))
    ue = jnp.where(valid, ue, pad_e)
    tab = jnp.stack([first_t, ubase, offs, ends])
    g = jnp.take(tab, ue, axis=1)
    ut = jnp.minimum(g[0] + (u - g[1]), nt - 1).astype(jnp.int32)
    urs = jnp.maximum(g[2] - ut * T, 0)
    ure = jnp.where(valid, jnp.minimum(g[3] - ut * T, T), 0)
    uf = jnp.concatenate(
        [jnp.ones((1,), jnp.int32), (ut[1:] != ut[:-1]).astype(jnp.int32)])
    return order, ue, urs.astype(jnp.int32), ure.astype(jnp.int32), ut, uf


def _make_sc_permute(n_tokens, d, reverse):
    rpw = n_tokens // NW
    mesh = plsc.VectorSubcoreMesh(core_axis_name="c", subcore_axis_name="s")

    @functools.partial(
        pl.kernel,
        out_type=jax.ShapeDtypeStruct((n_tokens, d), jnp.float32),
        mesh=mesh,
        scratch_types=[
            pltpu.VMEM((rpw,), jnp.int32),
            pltpu.VMEM((rpw, d), jnp.float32),
            pltpu.SemaphoreType.DMA,
        ],
    )
    def body(src_hbm, order_hbm, out_hbm, idx_v, rows_v, sem):
        wid = lax.axis_index("s") * NC + lax.axis_index("c")
        base = wid * rpw
        pltpu.sync_copy(order_hbm.at[pl.ds(base, rpw)], idx_v)
        if reverse:
            pltpu.sync_copy(src_hbm.at[pl.ds(base, rpw)], rows_v)
            pltpu.async_copy(rows_v, out_hbm.at[idx_v], sem).wait()
        else:
            pltpu.async_copy(src_hbm.at[idx_v], rows_v, sem).wait()
            pltpu.sync_copy(rows_v, out_hbm.at[pl.ds(base, rpw)])

    return body


def kernel(hidden_states, norm_w, gate_w, w1, w3, w2):
    b, s, d = hidden_states.shape
    n_experts, _, dff = w1.shape
    n_tokens = b * s
    nt = n_tokens // T
    n_units = nt + n_experts - 1
    x2d = hidden_states.reshape(n_tokens, d)

    xn, logits, selb = pl.pallas_call(
        _router_body,
        grid=(n_tokens // TA,),
        in_specs=[
            pl.BlockSpec((TA, d), lambda i: (i, 0)),
            pl.BlockSpec((1, d), lambda i: (0, 0)),
            pl.BlockSpec((n_experts, d), lambda i: (0, 0)),
        ],
        out_specs=[
            pl.BlockSpec((TA, d), lambda i: (i, 0)),
            pl.BlockSpec((TA, n_experts), lambda i: (i, 0)),
            pl.BlockSpec((TA, 128), lambda i: (i, 0)),
        ],
        out_shape=[
            jax.ShapeDtypeStruct((n_tokens, d), jnp.float32),
            jax.ShapeDtypeStruct((n_tokens, n_experts), jnp.float32),
            jax.ShapeDtypeStruct((n_tokens, 128), jnp.int32),
        ],
    )(x2d, norm_w.reshape(1, d), gate_w)
    sel = selb[:, 0]

    order, ue, urs, ure, ut, uf = _unit_metadata(sel, n_experts, n_tokens)

    xs = _make_sc_permute(n_tokens, d, reverse=False)(xn, order)

    grid_spec = pltpu.PrefetchScalarGridSpec(
        num_scalar_prefetch=5,
        grid=(n_units,),
        in_specs=[
            pl.BlockSpec((T, d), lambda u, ue, ut, urs, ure, uf: (ut[u], 0)),
            pl.BlockSpec(
                (1, d, dff), lambda u, ue, ut, urs, ure, uf: (ue[u], 0, 0)),
            pl.BlockSpec(
                (1, d, dff), lambda u, ue, ut, urs, ure, uf: (ue[u], 0, 0)),
            pl.BlockSpec(
                (1, dff, d), lambda u, ue, ut, urs, ure, uf: (ue[u], 0, 0)),
        ],
        out_specs=pl.BlockSpec(
            (T, d), lambda u, ue, ut, urs, ure, uf: (ut[u], 0)),
    )
    ys = pl.pallas_call(
        _ffn_body,
        grid_spec=grid_spec,
        out_shape=jax.ShapeDtypeStruct((n_tokens, d), jnp.float32),
    )(ue, ut, urs, ure, uf, xs, w1, w3, w2)

    final = _make_sc_permute(n_tokens, d, reverse=True)(ys, order)
    return final.reshape(b, s, d), logits


# speedup vs baseline: 1.2605x; 1.0054x over previous
import functools

import jax
import jax.numpy as jnp
from jax import lax
from jax.experimental import pallas as pl
from jax.experimental.pallas import tpu as pltpu
from jax.experimental.pallas import tpu_sc as plsc

EPS = 1e-6
T = 256
TA = 512
NC = 2
NS = 16
NW = NC * NS


def _router_body(x_ref, nw_ref, gw_ref, xn_ref, logits_ref, sel_ref):
    x = x_ref[...]
    v = jnp.mean(x * x, axis=-1, keepdims=True)
    xn = (x * lax.rsqrt(v + EPS)) * nw_ref[...]
    xn_ref[...] = xn
    logits = lax.dot_general(
        xn, gw_ref[...], (((1,), (1,)), ((), ())),
        preferred_element_type=jnp.float32)
    logits_ref[...] = logits
    e = logits.shape[-1]
    m = jnp.max(logits, axis=-1, keepdims=True)
    iota = lax.broadcasted_iota(jnp.int32, logits.shape, 1)
    am = jnp.min(jnp.where(logits == m, iota, e), axis=-1, keepdims=True)
    sel_ref[...] = jnp.broadcast_to(am, sel_ref.shape)


def _ffn_body(ue_ref, ut_ref, urs_ref, ure_ref, uf_ref,
              xs_ref, w1_ref, w3_ref, w2_ref, out_ref):
    u = pl.program_id(0)

    @pl.when(uf_ref[u] == 1)
    def _():
        out_ref[...] = jnp.zeros_like(out_ref)

    x = xs_ref[...]
    h1 = jnp.dot(x, w1_ref[0], preferred_element_type=jnp.float32)
    h3 = jnp.dot(x, w3_ref[0], preferred_element_type=jnp.float32)
    h = (h1 * jax.nn.sigmoid(h1)) * h3
    y = jnp.dot(h, w2_ref[0], preferred_element_type=jnp.float32)
    rows = lax.broadcasted_iota(jnp.int32, y.shape, 0)
    mask = (rows >= urs_ref[u]) & (rows < ure_ref[u])
    out_ref[...] += jnp.where(mask, y, 0.0)


def _unit_metadata(sel, n_experts, n_tokens):
    nt = n_tokens // T
    n_units = nt + n_experts - 1
    ssel, order = lax.sort_key_val(
        sel, jnp.arange(n_tokens, dtype=jnp.int32), is_stable=True)
    e_ids0 = jnp.arange(n_experts, dtype=jnp.int32)
    counts = jnp.sum(
        (ssel[:, None] == e_ids0[None, :]).astype(jnp.int32), axis=0)
    ends = jnp.cumsum(counts)
    offs = ends - counts
    first_t = offs // T
    last_t = jnp.where(counts > 0, (ends - 1) // T, first_t - 1)
    span = jnp.where(counts > 0, last_t - first_t + 1, 0)
    cum = jnp.cumsum(span)
    ubase = cum - span
    total = cum[-1]
    u = jnp.arange(n_units, dtype=jnp.int32)
    ue = jnp.minimum(
        jnp.sum((cum[None, :] <= u[:, None]).astype(jnp.int32), axis=1),
        n_experts - 1)
    valid = u < total
    e_ids = jnp.arange(n_experts, dtype=jnp.int32)
    pad_e = jnp.max(jnp.where(counts > 0, e_ids, 0))
    ue = jnp.where(valid, ue, pad_e)
    tab = jnp.stack([first_t, ubase, offs, ends]).astype(jnp.float32)
    ohu = (ue[:, None] == e_ids[None, :]).astype(jnp.float32)
    g = jnp.dot(ohu, tab.T).astype(jnp.int32)
    ut = jnp.minimum(g[:, 0] + (u - g[:, 1]), nt - 1).astype(jnp.int32)
    urs = jnp.maximum(g[:, 2] - ut * T, 0)
    ure = jnp.where(valid, jnp.minimum(g[:, 3] - ut * T, T), 0)
    uf = jnp.concatenate(
        [jnp.ones((1,), jnp.int32), (ut[1:] != ut[:-1]).astype(jnp.int32)])
    return order, ue, urs.astype(jnp.int32), ure.astype(jnp.int32), ut, uf


def _make_sc_permute(n_tokens, d, reverse):
    rpw = n_tokens // NW
    mesh = plsc.VectorSubcoreMesh(core_axis_name="c", subcore_axis_name="s")

    @functools.partial(
        pl.kernel,
        out_type=jax.ShapeDtypeStruct((n_tokens, d), jnp.float32),
        mesh=mesh,
        scratch_types=[
            pltpu.VMEM((rpw,), jnp.int32),
            pltpu.VMEM((rpw, d), jnp.float32),
            pltpu.SemaphoreType.DMA,
        ],
    )
    def body(src_hbm, order_hbm, out_hbm, idx_v, rows_v, sem):
        wid = lax.axis_index("s") * NC + lax.axis_index("c")
        base = wid * rpw
        pltpu.sync_copy(order_hbm.at[pl.ds(base, rpw)], idx_v)
        if reverse:
            pltpu.sync_copy(src_hbm.at[pl.ds(base, rpw)], rows_v)
            pltpu.async_copy(rows_v, out_hbm.at[idx_v], sem).wait()
        else:
            pltpu.async_copy(src_hbm.at[idx_v], rows_v, sem).wait()
            pltpu.sync_copy(rows_v, out_hbm.at[pl.ds(base, rpw)])

    return body


def kernel(hidden_states, norm_w, gate_w, w1, w3, w2):
    b, s, d = hidden_states.shape
    n_experts, _, dff = w1.shape
    n_tokens = b * s
    nt = n_tokens // T
    n_units = nt + n_experts - 1
    x2d = hidden_states.reshape(n_tokens, d)

    xn, logits, selb = pl.pallas_call(
        _router_body,
        grid=(n_tokens // TA,),
        in_specs=[
            pl.BlockSpec((TA, d), lambda i: (i, 0)),
            pl.BlockSpec((1, d), lambda i: (0, 0)),
            pl.BlockSpec((n_experts, d), lambda i: (0, 0)),
        ],
        out_specs=[
            pl.BlockSpec((TA, d), lambda i: (i, 0)),
            pl.BlockSpec((TA, n_experts), lambda i: (i, 0)),
            pl.BlockSpec((TA, 128), lambda i: (i, 0)),
        ],
        out_shape=[
            jax.ShapeDtypeStruct((n_tokens, d), jnp.float32),
            jax.ShapeDtypeStruct((n_tokens, n_experts), jnp.float32),
            jax.ShapeDtypeStruct((n_tokens, 128), jnp.int32),
        ],
    )(x2d, norm_w.reshape(1, d), gate_w)
    sel = selb[:, 0]

    order, ue, urs, ure, ut, uf = _unit_metadata(sel, n_experts, n_tokens)

    xs = _make_sc_permute(n_tokens, d, reverse=False)(xn, order)

    grid_spec = pltpu.PrefetchScalarGridSpec(
        num_scalar_prefetch=5,
        grid=(n_units,),
        in_specs=[
            pl.BlockSpec((T, d), lambda u, ue, ut, urs, ure, uf: (ut[u], 0)),
            pl.BlockSpec(
                (1, d, dff), lambda u, ue, ut, urs, ure, uf: (ue[u], 0, 0)),
            pl.BlockSpec(
                (1, d, dff), lambda u, ue, ut, urs, ure, uf: (ue[u], 0, 0)),
            pl.BlockSpec(
                (1, dff, d), lambda u, ue, ut, urs, ure, uf: (ue[u], 0, 0)),
        ],
        out_specs=pl.BlockSpec(
            (T, d), lambda u, ue, ut, urs, ure, uf: (ut[u], 0)),
    )
    ys = pl.pallas_call(
        _ffn_body,
        grid_spec=grid_spec,
        out_shape=jax.ShapeDtypeStruct((n_tokens, d), jnp.float32),
    )(ue, ut, urs, ure, uf, xs, w1, w3, w2)

    final = _make_sc_permute(n_tokens, d, reverse=True)(ys, order)
    return final.reshape(b, s, d), logits
